# baseline (speedup 1.0000x reference)
import jax
import jax.numpy as jnp
from jax import lax
from jax.experimental import pallas as pl
from jax.experimental.pallas import tpu as pltpu
from jax.experimental.pallas import tpu_sc as plsc

NUM_MONTH = 12
NUM_LOC = 100000
D = 128
B = 16384

_SC_ROWS = 12288
_TC_ROWS = B - _SC_ROWS

_NC = 2
_NS = 16
_NW = _NC * _NS
_BPW = _SC_ROWS // _NW
_NCHUNK = 4
_CHUNK = _BPW // _NCHUNK


def _sc_gather_body(y_hbm, table_hbm, out_hbm, idx_v, rows_v, gsems, wsem):
    wid = lax.axis_index("s") * _NC + lax.axis_index("c")
    base = wid * _BPW
    pltpu.sync_copy(y_hbm.at[1, pl.ds(base, _BPW)], idx_v)
    copies = []
    for j in range(_NCHUNK):
        sl = pl.ds(j * _CHUNK, _CHUNK)
        copies.append(
            pltpu.async_copy(table_hbm.at[idx_v.at[sl]], rows_v.at[sl],
                             gsems.at[j])
        )
    writes = []
    for j in range(_NCHUNK):
        sl = pl.ds(j * _CHUNK, _CHUNK)
        copies[j].wait()
        writes.append(
            pltpu.async_copy(rows_v.at[sl],
                             out_hbm.at[pl.ds(base + j * _CHUNK, _CHUNK)],
                             wsem)
        )
    for w in writes:
        w.wait()


def _sc_gather(loc_table, y):
    mesh = plsc.VectorSubcoreMesh(core_axis_name="c", subcore_axis_name="s")
    return pl.kernel(
        _sc_gather_body,
        out_type=jax.ShapeDtypeStruct((_SC_ROWS, D), jnp.float32),
        mesh=mesh,
        scratch_types=[
            pltpu.VMEM((_BPW,), jnp.int32),
            pltpu.VMEM((_BPW, D), jnp.float32),
            pltpu.SemaphoreType.DMA((_NCHUNK,)),
            pltpu.SemaphoreType.DMA,
        ],
        name="sc_gather",
    )(y, loc_table)


_TCB = 2048
_TC_TILES = _TC_ROWS // _TCB


def _tc_gather_body(idx_ref, table_ref, out_ref, sem):
    def issue(i, carry):
        r = idx_ref[1, 0, i]
        pltpu.make_async_copy(
            table_ref.at[pl.ds(r, 1), :], out_ref.at[pl.ds(i, 1), :], sem
        ).start()
        return carry

    lax.fori_loop(0, _TCB, issue, 0)
    pltpu.make_async_copy(table_ref.at[pl.ds(0, _TCB), :], out_ref, sem).wait()


def _tc_gather(y3, loc_table):
    tile0 = _SC_ROWS // _TCB
    return pl.pallas_call(
        _tc_gather_body,
        grid=(_TC_TILES,),
        in_specs=[
            pl.BlockSpec((2, 1, _TCB), lambda i: (0, 0, i + tile0),
                         memory_space=pltpu.SMEM),
            pl.BlockSpec(memory_space=pl.ANY),
        ],
        out_specs=pl.BlockSpec((_TCB, D), lambda i: (i, 0)),
        out_shape=jax.ShapeDtypeStruct((_TC_ROWS, D), jnp.float32),
        scratch_shapes=[pltpu.SemaphoreType.DMA],
        name="tc_gather",
    )(y3, loc_table)


_BB = 1024
_SC_TILES = _SC_ROWS // _BB
_TC_TILES_MLP = _TC_ROWS // _BB


def _mlp_body(month_ref, loc_ref, mt_ref, w1b_ref, b1_ref, w2_ref,
              b2_ref, out_ref):
    mt_proj = jnp.dot(mt_ref[...], w1b_ref[0], preferred_element_type=jnp.float32)
    labels = month_ref[0, 0, :]
    onehot = (labels[:, None]
              == lax.broadcasted_iota(jnp.int32, (_BB, NUM_MONTH), 1)
              ).astype(jnp.float32)
    h = (jnp.dot(onehot, mt_proj, preferred_element_type=jnp.float32)
         + jnp.dot(loc_ref[...], w1b_ref[1], preferred_element_type=jnp.float32)
         + b1_ref[...])
    h = h * jax.nn.sigmoid(h)
    out_ref[...] = (jnp.dot(h, w2_ref[...], preferred_element_type=jnp.float32)
                    + b2_ref[...])


def _mlp_body_skip_first(*refs):
    _mlp_body(*refs[1:])


def _tc_mlp_part(y3, loc_part, month_table, w1_split, b1r, b2r, W2, tile0,
                 n_tiles, out_prev):
    in_specs = [
        pl.BlockSpec((1, 1, _BB), lambda i: (0, 0, i + tile0)),
        pl.BlockSpec((_BB, D), lambda i: (i, 0)),
        pl.BlockSpec(month_table.shape, lambda i: (0, 0)),
        pl.BlockSpec(w1_split.shape, lambda i: (0, 0, 0)),
        pl.BlockSpec((1, D), lambda i: (0, 0)),
        pl.BlockSpec((D, D), lambda i: (0, 0)),
        pl.BlockSpec((1, D), lambda i: (0, 0)),
    ]
    args = (y3, loc_part, month_table, w1_split, b1r, W2, b2r)
    out_spec = pl.BlockSpec((_BB, D), lambda i: (i + tile0, 0))
    if out_prev is None:
        return pl.pallas_call(
            _mlp_body,
            grid=(n_tiles,),
            in_specs=in_specs,
            out_specs=out_spec,
            out_shape=jax.ShapeDtypeStruct((B, D), jnp.float32),
            name="tc_mlp_a",
        )(*args)
    return pl.pallas_call(
        _mlp_body_skip_first,
        grid=(n_tiles,),
        in_specs=[pl.BlockSpec(memory_space=pl.ANY)] + in_specs,
        out_specs=out_spec,
        out_shape=jax.ShapeDtypeStruct((B, D), jnp.float32),
        input_output_aliases={0: 0},
        name="tc_mlp_b",
    )(out_prev, *args)


@jax.jit
def _impl(y, month_table, loc_table, W1, b1, W2, b2):
    y = y.astype(jnp.int32)
    y3 = y.reshape(2, 1, B)
    w1_split = W1.reshape(2, D, D)
    b1r = b1.reshape(1, D)
    b2r = b2.reshape(1, D)
    loc_sc = _sc_gather(loc_table, y)
    loc_tc = _tc_gather(y3, loc_table)
    out = _tc_mlp_part(y3, loc_tc, month_table, w1_split, b1r, b2r, W2,
                       _SC_TILES, _TC_TILES_MLP, None)
    return _tc_mlp_part(y3, loc_sc, month_table, w1_split, b1r, b2r, W2,
                        0, _SC_TILES, out)


def kernel(y, month_table, loc_table, W1, b1, W2, b2):
    return _impl(y, month_table, loc_table, W1, b1, W2, b2)


# speedup vs baseline: 1.0399x; 1.0399x over previous
import jax
import jax.numpy as jnp
from jax import lax
from jax.experimental import pallas as pl
from jax.experimental.pallas import tpu as pltpu
from jax.experimental.pallas import tpu_sc as plsc

NUM_MONTH = 12
NUM_LOC = 100000
D = 128
B = 16384

_SC_ROWS = 12288
_TC_ROWS = B - _SC_ROWS

_NC = 2
_NS = 16
_NW = _NC * _NS
_BPW = _SC_ROWS // _NW
_NCHUNK = 4
_CHUNK = _BPW // _NCHUNK


def _sc_gather_body(y_hbm, table_hbm, out_hbm, idx_v, rows_v, gsems, wsem):
    wid = lax.axis_index("s") * _NC + lax.axis_index("c")
    base = wid * _BPW
    pltpu.sync_copy(y_hbm.at[1, pl.ds(base, _BPW)], idx_v)
    copies = []
    for j in range(_NCHUNK):
        sl = pl.ds(j * _CHUNK, _CHUNK)
        copies.append(
            pltpu.async_copy(table_hbm.at[idx_v.at[sl]], rows_v.at[sl],
                             gsems.at[j])
        )
    writes = []
    for j in range(_NCHUNK):
        sl = pl.ds(j * _CHUNK, _CHUNK)
        copies[j].wait()
        writes.append(
            pltpu.async_copy(rows_v.at[sl],
                             out_hbm.at[pl.ds(base + j * _CHUNK, _CHUNK)],
                             wsem)
        )
    for w in writes:
        w.wait()


def _sc_gather(loc_table, y):
    mesh = plsc.VectorSubcoreMesh(core_axis_name="c", subcore_axis_name="s")
    return pl.kernel(
        _sc_gather_body,
        out_type=jax.ShapeDtypeStruct((_SC_ROWS, D), jnp.float32),
        mesh=mesh,
        scratch_types=[
            pltpu.VMEM((_BPW,), jnp.int32),
            pltpu.VMEM((_BPW, D), jnp.float32),
            pltpu.SemaphoreType.DMA((_NCHUNK,)),
            pltpu.SemaphoreType.DMA,
        ],
        name="sc_gather",
    )(y, loc_table)


_TCB = 2048
_TC_TILES = _TC_ROWS // _TCB


def _tc_gather_body(idx_ref, table_ref, out_ref, sem):
    def issue(i, carry):
        r = idx_ref[1, 0, i]
        pltpu.make_async_copy(
            table_ref.at[pl.ds(r, 1), :], out_ref.at[pl.ds(i, 1), :], sem
        ).start()
        return carry

    lax.fori_loop(0, _TCB, issue, 0)
    pltpu.make_async_copy(table_ref.at[pl.ds(0, _TCB), :], out_ref, sem).wait()


def _tc_gather(y3, loc_table):
    tile0 = _SC_ROWS // _TCB
    return pl.pallas_call(
        _tc_gather_body,
        grid=(_TC_TILES,),
        in_specs=[
            pl.BlockSpec((2, 1, _TCB), lambda i: (0, 0, i + tile0),
                         memory_space=pltpu.SMEM),
            pl.BlockSpec(memory_space=pl.ANY),
        ],
        out_specs=pl.BlockSpec((_TCB, D), lambda i: (i, 0)),
        out_shape=jax.ShapeDtypeStruct((_TC_ROWS, D), jnp.float32),
        scratch_shapes=[pltpu.SemaphoreType.DMA],
        name="tc_gather",
    )(y3, loc_table)


_BB = 4096
_SC_TILES = _SC_ROWS // _BB
_TC_TILES_MLP = _TC_ROWS // _BB


def _mlp_body(month_ref, loc_ref, mt_ref, w1b_ref, b1_ref, w2_ref,
              b2_ref, out_ref):
    mt_proj = jnp.dot(mt_ref[...], w1b_ref[0], preferred_element_type=jnp.float32)
    labels = month_ref[0, 0, :]
    onehot = (labels[:, None]
              == lax.broadcasted_iota(jnp.int32, (_BB, NUM_MONTH), 1)
              ).astype(jnp.float32)
    h = (jnp.dot(onehot, mt_proj, preferred_element_type=jnp.float32)
         + jnp.dot(loc_ref[...], w1b_ref[1], preferred_element_type=jnp.float32)
         + b1_ref[...])
    h = h * jax.nn.sigmoid(h)
    out_ref[...] = (jnp.dot(h, w2_ref[...], preferred_element_type=jnp.float32)
                    + b2_ref[...])


def _mlp_body_skip_first(*refs):
    _mlp_body(*refs[1:])


def _tc_mlp_part(y3, loc_part, month_table, w1_split, b1r, b2r, W2, tile0,
                 n_tiles, out_prev):
    in_specs = [
        pl.BlockSpec((1, 1, _BB), lambda i: (0, 0, i + tile0)),
        pl.BlockSpec((_BB, D), lambda i: (i, 0)),
        pl.BlockSpec(month_table.shape, lambda i: (0, 0)),
        pl.BlockSpec(w1_split.shape, lambda i: (0, 0, 0)),
        pl.BlockSpec((1, D), lambda i: (0, 0)),
        pl.BlockSpec((D, D), lambda i: (0, 0)),
        pl.BlockSpec((1, D), lambda i: (0, 0)),
    ]
    args = (y3, loc_part, month_table, w1_split, b1r, W2, b2r)
    out_spec = pl.BlockSpec((_BB, D), lambda i: (i + tile0, 0))
    if out_prev is None:
        return pl.pallas_call(
            _mlp_body,
            grid=(n_tiles,),
            in_specs=in_specs,
            out_specs=out_spec,
            out_shape=jax.ShapeDtypeStruct((B, D), jnp.float32),
            name="tc_mlp_a",
        )(*args)
    return pl.pallas_call(
        _mlp_body_skip_first,
        grid=(n_tiles,),
        in_specs=[pl.BlockSpec(memory_space=pl.ANY)] + in_specs,
        out_specs=out_spec,
        out_shape=jax.ShapeDtypeStruct((B, D), jnp.float32),
        input_output_aliases={0: 0},
        name="tc_mlp_b",
    )(out_prev, *args)


@jax.jit
def _impl(y, month_table, loc_table, W1, b1, W2, b2):
    y = y.astype(jnp.int32)
    y3 = y.reshape(2, 1, B)
    w1_split = W1.reshape(2, D, D)
    b1r = b1.reshape(1, D)
    b2r = b2.reshape(1, D)
    loc_sc = _sc_gather(loc_table, y)
    loc_tc = _tc_gather(y3, loc_table)
    out = _tc_mlp_part(y3, loc_tc, month_table, w1_split, b1r, b2r, W2,
                       _SC_TILES, _TC_TILES_MLP, None)
    return _tc_mlp_part(y3, loc_sc, month_table, w1_split, b1r, b2r, W2,
                        0, _SC_TILES, out)


def kernel(y, month_table, loc_table, W1, b1, W2, b2):
    return _impl(y, month_table, loc_table, W1, b1, W2, b2)
